# baseline (speedup 1.0000x reference)
import functools

import jax
import jax.numpy as jnp
from jax import lax
from jax.experimental import pallas as pl
from jax.experimental.pallas import tpu as pltpu
from jax.experimental.pallas import tpu_sc as plsc

N = 10000
E = 320000
D = 128
H = 16
C = 40

NC = 2
NS = 16
NW = NC * NS
EPW = E // NW
EB = 80
NB = EPW // EB
GW = 32
ZR = 624
ZTAIL = N - NS * ZR

_ROWBLK = 1000


def _tc_in_body(x_ref, w1t_ref, b1_ref, xn_ref, nrm_ref):
    h = jnp.dot(x_ref[...], w1t_ref[...], preferred_element_type=jnp.float32)
    h = jnp.maximum(h + b1_ref[...], 0.0)
    n2 = jnp.sum(h * h, axis=1, keepdims=True)
    norm = jnp.sqrt(n2)
    rnorm = 1.0 / jnp.maximum(norm, 1e-12)
    xn_ref[...] = h * rnorm
    nrm_ref[...] = norm


def _tc_mid_body(pa_ref, pb_ref, xn_ref, nrm_ref, xno_ref, nrmo_ref):
    pa = pa_ref[...]
    pb = pb_ref[...]
    xn = xn_ref[...]
    h_prev = xn * nrm_ref[...]
    c_prev = jnp.sum(xn * xn, axis=1, keepdims=True)
    es = jnp.exp(c_prev - 1.0)
    num = pa[:, 0:H] + pb[:, 0:H] + h_prev * es
    den = (jnp.sum(pa[:, H:GW], axis=1, keepdims=True)
           + jnp.sum(pb[:, H:GW], axis=1, keepdims=True) + es)
    h = num / den
    n2 = jnp.sum(h * h, axis=1, keepdims=True)
    norm = jnp.sqrt(n2)
    rnorm = 1.0 / jnp.maximum(norm, 1e-12)
    xno_ref[...] = h * rnorm
    nrmo_ref[...] = norm


def _tc_out_body(pa_ref, pb_ref, xn_ref, nrm_ref, beta_ref, w2t_ref, b2_ref,
                 out_ref):
    beta = beta_ref[0, 0]
    pa = pa_ref[...]
    pb = pb_ref[...]
    xn = xn_ref[...]
    h_prev = xn * nrm_ref[...]
    c_prev = jnp.sum(xn * xn, axis=1, keepdims=True)
    es = jnp.exp(beta * c_prev - jnp.abs(beta))
    num = pa[:, 0:H] + pb[:, 0:H] + h_prev * es
    den = (jnp.sum(pa[:, H:GW], axis=1, keepdims=True)
           + jnp.sum(pb[:, H:GW], axis=1, keepdims=True) + es)
    h = num / den
    logits = jnp.dot(h, w2t_ref[...], preferred_element_type=jnp.float32)
    logits = logits + b2_ref[...]
    m = jnp.max(logits, axis=1, keepdims=True)
    lse = jnp.log(jnp.sum(jnp.exp(logits - m), axis=1, keepdims=True)) + m
    out_ref[...] = logits - lse


def _tc_in(x, w1t, b1):
    return pl.pallas_call(
        _tc_in_body,
        grid=(N // _ROWBLK,),
        in_specs=[
            pl.BlockSpec((_ROWBLK, D), lambda i: (i, 0)),
            pl.BlockSpec((D, H), lambda i: (0, 0)),
            pl.BlockSpec((1, H), lambda i: (0, 0)),
        ],
        out_specs=[
            pl.BlockSpec((_ROWBLK, H), lambda i: (i, 0)),
            pl.BlockSpec((_ROWBLK, 1), lambda i: (i, 0)),
        ],
        out_shape=[
            jax.ShapeDtypeStruct((N, H), jnp.float32),
            jax.ShapeDtypeStruct((N, 1), jnp.float32),
        ],
    )(x, w1t, b1)


def _tc_mid(pa, pb, xn, nrm):
    return pl.pallas_call(
        _tc_mid_body,
        grid=(N // _ROWBLK,),
        in_specs=[
            pl.BlockSpec((_ROWBLK, GW), lambda i: (i, 0)),
            pl.BlockSpec((_ROWBLK, GW), lambda i: (i, 0)),
            pl.BlockSpec((_ROWBLK, H), lambda i: (i, 0)),
            pl.BlockSpec((_ROWBLK, 1), lambda i: (i, 0)),
        ],
        out_specs=[
            pl.BlockSpec((_ROWBLK, H), lambda i: (i, 0)),
            pl.BlockSpec((_ROWBLK, 1), lambda i: (i, 0)),
        ],
        out_shape=[
            jax.ShapeDtypeStruct((N, H), jnp.float32),
            jax.ShapeDtypeStruct((N, 1), jnp.float32),
        ],
    )(pa, pb, xn, nrm)


def _tc_out(pa, pb, xn, nrm, beta, w2t, b2):
    return pl.pallas_call(
        _tc_out_body,
        grid=(N // _ROWBLK,),
        in_specs=[
            pl.BlockSpec((_ROWBLK, GW), lambda i: (i, 0)),
            pl.BlockSpec((_ROWBLK, GW), lambda i: (i, 0)),
            pl.BlockSpec((_ROWBLK, H), lambda i: (i, 0)),
            pl.BlockSpec((_ROWBLK, 1), lambda i: (i, 0)),
            pl.BlockSpec((1, 1), lambda i: (0, 0)),
            pl.BlockSpec((H, C), lambda i: (0, 0)),
            pl.BlockSpec((1, C), lambda i: (0, 0)),
        ],
        out_specs=pl.BlockSpec((_ROWBLK, C), lambda i: (i, 0)),
        out_shape=jax.ShapeDtypeStruct((N, C), jnp.float32),
    )(pa, pb, xn, nrm, beta, w2t, b2)


def _sc_prop(xn, nrm, src2d, dst2d, bvec):
    mesh = plsc.VectorSubcoreMesh(
        core_axis_name="c", subcore_axis_name="s",
        num_cores=NC, num_subcores=NS)

    @functools.partial(
        pl.kernel,
        out_type=jax.ShapeDtypeStruct((NC, N, GW), jnp.float32),
        mesh=mesh,
        scratch_types=[
            pltpu.VMEM((NB, EB), jnp.int32),
            pltpu.VMEM((NB, EB), jnp.int32),
            pltpu.VMEM((N,), jnp.float32),
            pltpu.VMEM((EB, H), jnp.float32),
            pltpu.VMEM((EB, H), jnp.float32),
            pltpu.VMEM((EB, H), jnp.float32),
            pltpu.VMEM((EB, H), jnp.float32),
            pltpu.VMEM((EB, GW), jnp.float32),
            pltpu.VMEM((EB, GW), jnp.float32),
            pltpu.VMEM((16,), jnp.float32),
            pltpu.VMEM((ZR, GW), jnp.float32),
            pltpu.VMEM_SHARED((N, GW), jnp.float32),
            pltpu.SemaphoreType.DMA,
            pltpu.SemaphoreType.DMA,
            pltpu.SemaphoreType.DMA,
            pltpu.SemaphoreType.DMA,
            pltpu.SemaphoreType.DMA,
            pltpu.SemaphoreType.DMA,
        ],
        compiler_params=pltpu.CompilerParams(
            needs_layout_passes=False, use_tc_tiling_on_sc=False),
    )
    def k(xn_hbm, nrm_hbm, src_hbm, dst_hbm, bvec_hbm, out_hbm,
          src_v, dst_v, nrm_v, gs0, gs1, gd0, gd1, ct0, ct1, b_v, z_v, acc_sh,
          sgs0, sgs1, sgd0, sgd1, sct0, sct1):
        cid = lax.axis_index("c")
        sid = lax.axis_index("s")
        wid = cid * NS + sid

        zero16 = jnp.zeros((16,), jnp.float32)

        def zbody(i, carry):
            z_v[i, 0:16] = zero16
            z_v[i, 16:32] = zero16
            return carry

        lax.fori_loop(0, ZR, zbody, 0)
        pltpu.sync_copy(z_v, acc_sh.at[pl.ds(sid * ZR, ZR)])

        @pl.when(sid == NS - 1)
        def _():
            pltpu.sync_copy(z_v.at[pl.ds(0, ZTAIL)],
                            acc_sh.at[pl.ds(NS * ZR, ZTAIL)])

        pltpu.sync_copy(bvec_hbm, b_v)
        pltpu.sync_copy(nrm_hbm, nrm_v)
        pltpu.sync_copy(src_hbm.at[pl.ds(wid * NB, NB)], src_v)
        pltpu.sync_copy(dst_hbm.at[pl.ds(wid * NB, NB)], dst_v)
        plsc.subcore_barrier()

        bv16 = b_v[...]
        beta = bv16[0]
        ab = bv16[1]
        iota16 = lax.iota(jnp.int32, 16)
        diagcols = [(iota16 + kk) & 15 for kk in range(H)]
        ecols = iota16 + H
        slots = ((gs0, gd0, sgs0, sgd0), (gs1, gd1, sgs1, sgd1))
        cts = ((ct0, sct0), (ct1, sct1))

        for j in range(EB):
            ct0[j, H:GW] = zero16
            ct1[j, H:GW] = zero16

        def issue(bi, slot):
            gs, gd, sgs, sgd = slots[slot]
            pltpu.async_copy(xn_hbm.at[src_v.at[bi]], gs, sgs)
            pltpu.async_copy(xn_hbm.at[dst_v.at[bi]], gd, sgd)

        def wait_slot(slot):
            gs, gd, sgs, sgd = slots[slot]
            pltpu.make_async_copy(xn_hbm.at[pl.ds(0, EB)], gs, sgs).wait()
            pltpu.make_async_copy(xn_hbm.at[pl.ds(0, EB)], gd, sgd).wait()

        def scatter_start(bi, slot):
            ct, sct = cts[slot]
            pltpu.async_copy(ct, acc_sh.at[dst_v.at[bi]], sct, add=True)

        def scatter_wait(slot):
            ct, sct = cts[slot]
            pltpu.make_async_copy(ct, acc_sh.at[pl.ds(0, EB)], sct).wait()

        def compute_block(bi, slot):
            gs, gd, _, _ = slots[slot]
            ct_v, _ = cts[slot]
            for grp in range(EB // 16):
                rows = iota16 + (grp * 16)
                adiag = []
                terms = []
                for kk in range(H):
                    a = plsc.load_gather(gs, [rows, diagcols[kk]])
                    b = plsc.load_gather(gd, [rows, diagcols[kk]])
                    adiag.append(a)
                    terms.append(a * b)
                while len(terms) > 1:
                    terms = [terms[i] + terms[i + 1]
                             for i in range(0, len(terms), 2)]
                score = terms[0]
                s16 = src_v[bi, pl.ds(grp * 16, 16)]
                nsrc = plsc.load_gather(nrm_v, [s16])
                e16 = jnp.exp(beta * score - ab)
                sc16 = e16 * nsrc
                for kk in range(H):
                    plsc.store_scatter(ct_v, [rows, diagcols[kk]],
                                       adiag[kk] * sc16)
                plsc.store_scatter(ct_v, [rows, ecols], e16)

        issue(0, 0)

        def pair(g, carry):
            b0 = 2 * g
            issue(b0 + 1, 1)
            wait_slot(0)

            @pl.when(g > 0)
            def _():
                scatter_wait(0)

            compute_block(b0, 0)
            scatter_start(b0, 0)
            issue(b0 + 2, 0)
            wait_slot(1)

            @pl.when(g > 0)
            def _():
                scatter_wait(1)

            compute_block(b0 + 1, 1)
            scatter_start(b0 + 1, 1)
            return carry

        lax.fori_loop(0, (NB - 1) // 2, pair, 0)
        wait_slot(0)
        scatter_wait(0)
        compute_block(NB - 1, 0)
        scatter_start(NB - 1, 0)
        scatter_wait(0)
        scatter_wait(1)

        plsc.subcore_barrier()
        pltpu.sync_copy(acc_sh.at[pl.ds(sid * ZR, ZR)],
                        out_hbm.at[cid, pl.ds(sid * ZR, ZR)])

        @pl.when(sid == NS - 1)
        def _():
            pltpu.sync_copy(acc_sh.at[pl.ds(NS * ZR, ZTAIL)],
                            out_hbm.at[cid, pl.ds(NS * ZR, ZTAIL)])

    return k(xn, nrm, src2d, dst2d, bvec)


def kernel(x, edge_index, W1, b1, W2, b2, beta2):
    x = x.astype(jnp.float32)
    ei = edge_index.astype(jnp.int32)
    src2d = ei[0].reshape(NW * NB, EB)
    dst2d = ei[1].reshape(NW * NB, EB)
    w1t = W1.astype(jnp.float32).T
    w2t = W2.astype(jnp.float32).T
    b1r = b1.astype(jnp.float32).reshape(1, H)
    b2r = b2.astype(jnp.float32).reshape(1, C)
    beta = beta2.astype(jnp.float32)[0]

    xn0, nrm0 = _tc_in(x, w1t, b1r)
    bvec1 = jnp.zeros((16,), jnp.float32).at[0].set(1.0).at[1].set(1.0)
    parts1 = _sc_prop(xn0, nrm0.reshape(N), src2d, dst2d, bvec1)
    xn1, nrm1 = _tc_mid(parts1[0], parts1[1], xn0, nrm0)
    bvec2 = jnp.zeros((16,), jnp.float32).at[0].set(beta).at[1].set(jnp.abs(beta))
    parts2 = _sc_prop(xn1, nrm1.reshape(N), src2d, dst2d, bvec2)
    return _tc_out(parts2[0], parts2[1], xn1, nrm1,
                   beta2.astype(jnp.float32).reshape(1, 1), w2t, b2r)


# speedup vs baseline: 1.2265x; 1.2265x over previous
import functools

import jax
import jax.numpy as jnp
from jax import lax
from jax.experimental import pallas as pl
from jax.experimental.pallas import tpu as pltpu
from jax.experimental.pallas import tpu_sc as plsc

N = 10000
E = 320000
D = 128
H = 16
C = 40

NC = 2
NS = 16
NW = NC * NS
EPW = E // NW
EB = 80
NB = EPW // EB
GW = 32
ZR = 624
ZTAIL = N - NS * ZR

_ROWBLK = 1000


def _tc_in_body(x_ref, w1t_ref, b1_ref, xn_ref, nrm_ref):
    h = jnp.dot(x_ref[...], w1t_ref[...], preferred_element_type=jnp.float32)
    h = jnp.maximum(h + b1_ref[...], 0.0)
    n2 = jnp.sum(h * h, axis=1, keepdims=True)
    norm = jnp.sqrt(n2)
    rnorm = 1.0 / jnp.maximum(norm, 1e-12)
    xn_ref[...] = h * rnorm
    nrm_ref[...] = norm


def _tc_mid_body(pa_ref, pb_ref, xn_ref, nrm_ref, xno_ref, nrmo_ref):
    pa = pa_ref[...]
    pb = pb_ref[...]
    xn = xn_ref[...]
    h_prev = xn * nrm_ref[...]
    c_prev = jnp.sum(xn * xn, axis=1, keepdims=True)
    es = jnp.exp(c_prev - 1.0)
    num = pa[:, 0:H] + pb[:, 0:H] + h_prev * es
    den = (jnp.sum(pa[:, H:GW], axis=1, keepdims=True)
           + jnp.sum(pb[:, H:GW], axis=1, keepdims=True) + es)
    h = num / den
    n2 = jnp.sum(h * h, axis=1, keepdims=True)
    norm = jnp.sqrt(n2)
    rnorm = 1.0 / jnp.maximum(norm, 1e-12)
    xno_ref[...] = h * rnorm
    nrmo_ref[...] = norm


def _tc_out_body(pa_ref, pb_ref, xn_ref, nrm_ref, beta_ref, w2t_ref, b2_ref,
                 out_ref):
    beta = beta_ref[0, 0]
    pa = pa_ref[...]
    pb = pb_ref[...]
    xn = xn_ref[...]
    h_prev = xn * nrm_ref[...]
    c_prev = jnp.sum(xn * xn, axis=1, keepdims=True)
    es = jnp.exp(beta * c_prev - jnp.abs(beta))
    num = pa[:, 0:H] + pb[:, 0:H] + h_prev * es
    den = (jnp.sum(pa[:, H:GW], axis=1, keepdims=True)
           + jnp.sum(pb[:, H:GW], axis=1, keepdims=True) + es)
    h = num / den
    logits = jnp.dot(h, w2t_ref[...], preferred_element_type=jnp.float32)
    logits = logits + b2_ref[...]
    m = jnp.max(logits, axis=1, keepdims=True)
    lse = jnp.log(jnp.sum(jnp.exp(logits - m), axis=1, keepdims=True)) + m
    out_ref[...] = logits - lse


def _tc_in(x, w1t, b1):
    return pl.pallas_call(
        _tc_in_body,
        grid=(N // _ROWBLK,),
        in_specs=[
            pl.BlockSpec((_ROWBLK, D), lambda i: (i, 0)),
            pl.BlockSpec((D, H), lambda i: (0, 0)),
            pl.BlockSpec((1, H), lambda i: (0, 0)),
        ],
        out_specs=[
            pl.BlockSpec((_ROWBLK, H), lambda i: (i, 0)),
            pl.BlockSpec((_ROWBLK, 1), lambda i: (i, 0)),
        ],
        out_shape=[
            jax.ShapeDtypeStruct((N, H), jnp.float32),
            jax.ShapeDtypeStruct((N, 1), jnp.float32),
        ],
    )(x, w1t, b1)


def _tc_mid(pa, pb, xn, nrm):
    return pl.pallas_call(
        _tc_mid_body,
        grid=(N // _ROWBLK,),
        in_specs=[
            pl.BlockSpec((_ROWBLK, GW), lambda i: (i, 0)),
            pl.BlockSpec((_ROWBLK, GW), lambda i: (i, 0)),
            pl.BlockSpec((_ROWBLK, H), lambda i: (i, 0)),
            pl.BlockSpec((_ROWBLK, 1), lambda i: (i, 0)),
        ],
        out_specs=[
            pl.BlockSpec((_ROWBLK, H), lambda i: (i, 0)),
            pl.BlockSpec((_ROWBLK, 1), lambda i: (i, 0)),
        ],
        out_shape=[
            jax.ShapeDtypeStruct((N, H), jnp.float32),
            jax.ShapeDtypeStruct((N, 1), jnp.float32),
        ],
    )(pa, pb, xn, nrm)


def _tc_out(pa, pb, xn, nrm, beta, w2t, b2):
    return pl.pallas_call(
        _tc_out_body,
        grid=(N // _ROWBLK,),
        in_specs=[
            pl.BlockSpec((_ROWBLK, GW), lambda i: (i, 0)),
            pl.BlockSpec((_ROWBLK, GW), lambda i: (i, 0)),
            pl.BlockSpec((_ROWBLK, H), lambda i: (i, 0)),
            pl.BlockSpec((_ROWBLK, 1), lambda i: (i, 0)),
            pl.BlockSpec((1, 1), lambda i: (0, 0)),
            pl.BlockSpec((H, C), lambda i: (0, 0)),
            pl.BlockSpec((1, C), lambda i: (0, 0)),
        ],
        out_specs=pl.BlockSpec((_ROWBLK, C), lambda i: (i, 0)),
        out_shape=jax.ShapeDtypeStruct((N, C), jnp.float32),
    )(pa, pb, xn, nrm, beta, w2t, b2)


def _sc_prop(xn, nrm, src2d, dst2d, bvec):
    mesh = plsc.VectorSubcoreMesh(
        core_axis_name="c", subcore_axis_name="s",
        num_cores=NC, num_subcores=NS)

    @functools.partial(
        pl.kernel,
        out_type=jax.ShapeDtypeStruct((NC, N, GW), jnp.float32),
        mesh=mesh,
        scratch_types=[
            pltpu.VMEM((NB, EB), jnp.int32),
            pltpu.VMEM((NB, EB), jnp.int32),
            pltpu.VMEM((N,), jnp.float32),
            pltpu.VMEM((EB, H), jnp.float32),
            pltpu.VMEM((EB, H), jnp.float32),
            pltpu.VMEM((EB, H), jnp.float32),
            pltpu.VMEM((EB, H), jnp.float32),
            pltpu.VMEM((EB, GW), jnp.float32),
            pltpu.VMEM((EB, GW), jnp.float32),
            pltpu.VMEM((16,), jnp.float32),
            pltpu.VMEM((ZR, GW), jnp.float32),
            pltpu.VMEM_SHARED((N, GW), jnp.float32),
            pltpu.VMEM_SHARED((N, H), jnp.float32),
            pltpu.SemaphoreType.DMA,
            pltpu.SemaphoreType.DMA,
            pltpu.SemaphoreType.DMA,
            pltpu.SemaphoreType.DMA,
            pltpu.SemaphoreType.DMA,
            pltpu.SemaphoreType.DMA,
        ],
        compiler_params=pltpu.CompilerParams(
            needs_layout_passes=False, use_tc_tiling_on_sc=False),
    )
    def k(xn_hbm, nrm_hbm, src_hbm, dst_hbm, bvec_hbm, out_hbm,
          src_v, dst_v, nrm_v, gs0, gs1, gd0, gd1, ct0, ct1, b_v, z_v, acc_sh,
          xn_sh, sgs0, sgs1, sgd0, sgd1, sct0, sct1):
        cid = lax.axis_index("c")
        sid = lax.axis_index("s")
        wid = cid * NS + sid

        zero16 = jnp.zeros((16,), jnp.float32)

        def zbody(i, carry):
            z_v[i, 0:16] = zero16
            z_v[i, 16:32] = zero16
            return carry

        lax.fori_loop(0, ZR, zbody, 0)
        pltpu.sync_copy(z_v, acc_sh.at[pl.ds(sid * ZR, ZR)])

        @pl.when(sid == NS - 1)
        def _():
            pltpu.sync_copy(z_v.at[pl.ds(0, ZTAIL)],
                            acc_sh.at[pl.ds(NS * ZR, ZTAIL)])

        pltpu.sync_copy(bvec_hbm, b_v)
        pltpu.sync_copy(nrm_hbm, nrm_v)
        pltpu.sync_copy(src_hbm.at[pl.ds(wid * NB, NB)], src_v)
        pltpu.sync_copy(dst_hbm.at[pl.ds(wid * NB, NB)], dst_v)
        pltpu.sync_copy(xn_hbm.at[pl.ds(sid * ZR, ZR)],
                        xn_sh.at[pl.ds(sid * ZR, ZR)])

        @pl.when(sid == NS - 1)
        def _():
            pltpu.sync_copy(xn_hbm.at[pl.ds(NS * ZR, ZTAIL)],
                            xn_sh.at[pl.ds(NS * ZR, ZTAIL)])

        plsc.subcore_barrier()

        bv16 = b_v[...]
        beta = bv16[0]
        ab = bv16[1]
        iota16 = lax.iota(jnp.int32, 16)
        diagcols = [(iota16 + kk) & 15 for kk in range(H)]
        ecols = iota16 + H
        slots = ((gs0, gd0, sgs0, sgd0), (gs1, gd1, sgs1, sgd1))
        cts = ((ct0, sct0), (ct1, sct1))

        for j in range(EB):
            ct0[j, H:GW] = zero16
            ct1[j, H:GW] = zero16

        def issue(bi, slot):
            gs, gd, sgs, sgd = slots[slot]
            pltpu.async_copy(xn_sh.at[src_v.at[bi]], gs, sgs)
            pltpu.async_copy(xn_sh.at[dst_v.at[bi]], gd, sgd)

        def wait_slot(slot):
            gs, gd, sgs, sgd = slots[slot]
            pltpu.make_async_copy(xn_hbm.at[pl.ds(0, EB)], gs, sgs).wait()
            pltpu.make_async_copy(xn_hbm.at[pl.ds(0, EB)], gd, sgd).wait()

        def scatter_start(bi, slot):
            ct, sct = cts[slot]
            pltpu.async_copy(ct, acc_sh.at[dst_v.at[bi]], sct, add=True)

        def scatter_wait(slot):
            ct, sct = cts[slot]
            pltpu.make_async_copy(ct, acc_sh.at[pl.ds(0, EB)], sct).wait()

        def compute_block(bi, slot):
            gs, gd, _, _ = slots[slot]
            ct_v, _ = cts[slot]
            for grp in range(EB // 16):
                rows = iota16 + (grp * 16)
                adiag = []
                terms = []
                for kk in range(H):
                    a = plsc.load_gather(gs, [rows, diagcols[kk]])
                    b = plsc.load_gather(gd, [rows, diagcols[kk]])
                    adiag.append(a)
                    terms.append(a * b)
                while len(terms) > 1:
                    terms = [terms[i] + terms[i + 1]
                             for i in range(0, len(terms), 2)]
                score = terms[0]
                s16 = src_v[bi, pl.ds(grp * 16, 16)]
                nsrc = plsc.load_gather(nrm_v, [s16])
                e16 = jnp.exp(beta * score - ab)
                sc16 = e16 * nsrc
                for kk in range(H):
                    plsc.store_scatter(ct_v, [rows, diagcols[kk]],
                                       adiag[kk] * sc16)
                plsc.store_scatter(ct_v, [rows, ecols], e16)

        issue(0, 0)

        def pair(g, carry):
            b0 = 2 * g
            issue(b0 + 1, 1)
            wait_slot(0)

            @pl.when(g > 0)
            def _():
                scatter_wait(0)

            compute_block(b0, 0)
            scatter_start(b0, 0)
            issue(b0 + 2, 0)
            wait_slot(1)

            @pl.when(g > 0)
            def _():
                scatter_wait(1)

            compute_block(b0 + 1, 1)
            scatter_start(b0 + 1, 1)
            return carry

        lax.fori_loop(0, (NB - 1) // 2, pair, 0)
        wait_slot(0)
        scatter_wait(0)
        compute_block(NB - 1, 0)
        scatter_start(NB - 1, 0)
        scatter_wait(0)
        scatter_wait(1)

        plsc.subcore_barrier()
        pltpu.sync_copy(acc_sh.at[pl.ds(sid * ZR, ZR)],
                        out_hbm.at[cid, pl.ds(sid * ZR, ZR)])

        @pl.when(sid == NS - 1)
        def _():
            pltpu.sync_copy(acc_sh.at[pl.ds(NS * ZR, ZTAIL)],
                            out_hbm.at[cid, pl.ds(NS * ZR, ZTAIL)])

    return k(xn, nrm, src2d, dst2d, bvec)


def kernel(x, edge_index, W1, b1, W2, b2, beta2):
    x = x.astype(jnp.float32)
    ei = edge_index.astype(jnp.int32)
    src2d = ei[0].reshape(NW * NB, EB)
    dst2d = ei[1].reshape(NW * NB, EB)
    w1t = W1.astype(jnp.float32).T
    w2t = W2.astype(jnp.float32).T
    b1r = b1.astype(jnp.float32).reshape(1, H)
    b2r = b2.astype(jnp.float32).reshape(1, C)
    beta = beta2.astype(jnp.float32)[0]

    xn0, nrm0 = _tc_in(x, w1t, b1r)
    bvec1 = jnp.zeros((16,), jnp.float32).at[0].set(1.0).at[1].set(1.0)
    parts1 = _sc_prop(xn0, nrm0.reshape(N), src2d, dst2d, bvec1)
    xn1, nrm1 = _tc_mid(parts1[0], parts1[1], xn0, nrm0)
    bvec2 = jnp.zeros((16,), jnp.float32).at[0].set(beta).at[1].set(jnp.abs(beta))
    parts2 = _sc_prop(xn1, nrm1.reshape(N), src2d, dst2d, bvec2)
    return _tc_out(parts2[0], parts2[1], xn1, nrm1,
                   beta2.astype(jnp.float32).reshape(1, 1), w2t, b2r)


# speedup vs baseline: 1.3023x; 1.0618x over previous
import functools

import jax
import jax.numpy as jnp
from jax import lax
from jax.experimental import pallas as pl
from jax.experimental.pallas import tpu as pltpu
from jax.experimental.pallas import tpu_sc as plsc

N = 10000
E = 320000
D = 128
H = 16
C = 40

NC = 2
NS = 16
NW = NC * NS
EPW = E // NW
EB = 80
NB = EPW // EB
GW = 32
ZR = 624
ZTAIL = N - NS * ZR

_ROWBLK = 1000


def _tc_in_body(x_ref, w1t_ref, b1_ref, xn_ref, nrm_ref):
    h = jnp.dot(x_ref[...], w1t_ref[...], preferred_element_type=jnp.float32)
    h = jnp.maximum(h + b1_ref[...], 0.0)
    n2 = jnp.sum(h * h, axis=1, keepdims=True)
    norm = jnp.sqrt(n2)
    rnorm = 1.0 / jnp.maximum(norm, 1e-12)
    xn_ref[...] = h * rnorm
    nrm_ref[...] = norm


def _tc_mid_body(parts_ref, xn_ref, nrm_ref, xno_ref, nrmo_ref):
    pa = parts_ref[0]
    pb = parts_ref[1]
    xn = xn_ref[...]
    h_prev = xn * nrm_ref[...]
    c_prev = jnp.sum(xn * xn, axis=1, keepdims=True)
    es = jnp.exp(c_prev - 1.0)
    num = pa[:, 0:H] + pb[:, 0:H] + h_prev * es
    den = (jnp.sum(pa[:, H:GW], axis=1, keepdims=True)
           + jnp.sum(pb[:, H:GW], axis=1, keepdims=True) + es)
    h = num / den
    n2 = jnp.sum(h * h, axis=1, keepdims=True)
    norm = jnp.sqrt(n2)
    rnorm = 1.0 / jnp.maximum(norm, 1e-12)
    xno_ref[...] = h * rnorm
    nrmo_ref[...] = norm


def _tc_out_body(parts_ref, xn_ref, nrm_ref, beta_ref, w2t_ref, b2_ref,
                 out_ref):
    beta = beta_ref[0, 0]
    pa = parts_ref[0]
    pb = parts_ref[1]
    xn = xn_ref[...]
    h_prev = xn * nrm_ref[...]
    c_prev = jnp.sum(xn * xn, axis=1, keepdims=True)
    es = jnp.exp(beta * c_prev - jnp.abs(beta))
    num = pa[:, 0:H] + pb[:, 0:H] + h_prev * es
    den = (jnp.sum(pa[:, H:GW], axis=1, keepdims=True)
           + jnp.sum(pb[:, H:GW], axis=1, keepdims=True) + es)
    h = num / den
    logits = jnp.dot(h, w2t_ref[...], preferred_element_type=jnp.float32)
    logits = logits + b2_ref[...]
    m = jnp.max(logits, axis=1, keepdims=True)
    lse = jnp.log(jnp.sum(jnp.exp(logits - m), axis=1, keepdims=True)) + m
    out_ref[...] = logits - lse


def _tc_in(x, w1t, b1):
    return pl.pallas_call(
        _tc_in_body,
        grid=(N // _ROWBLK,),
        in_specs=[
            pl.BlockSpec((_ROWBLK, D), lambda i: (i, 0)),
            pl.BlockSpec((D, H), lambda i: (0, 0)),
            pl.BlockSpec((1, H), lambda i: (0, 0)),
        ],
        out_specs=[
            pl.BlockSpec((_ROWBLK, H), lambda i: (i, 0)),
            pl.BlockSpec((_ROWBLK, 1), lambda i: (i, 0)),
        ],
        out_shape=[
            jax.ShapeDtypeStruct((N, H), jnp.float32),
            jax.ShapeDtypeStruct((N, 1), jnp.float32),
        ],
    )(x, w1t, b1)


def _tc_mid(parts, xn, nrm):
    return pl.pallas_call(
        _tc_mid_body,
        grid=(N // _ROWBLK,),
        in_specs=[
            pl.BlockSpec((NC, _ROWBLK, GW), lambda i: (0, i, 0)),
            pl.BlockSpec((_ROWBLK, H), lambda i: (i, 0)),
            pl.BlockSpec((_ROWBLK, 1), lambda i: (i, 0)),
        ],
        out_specs=[
            pl.BlockSpec((_ROWBLK, H), lambda i: (i, 0)),
            pl.BlockSpec((_ROWBLK, 1), lambda i: (i, 0)),
        ],
        out_shape=[
            jax.ShapeDtypeStruct((N, H), jnp.float32),
            jax.ShapeDtypeStruct((N, 1), jnp.float32),
        ],
    )(parts, xn, nrm)


def _tc_out(parts, xn, nrm, beta, w2t, b2):
    return pl.pallas_call(
        _tc_out_body,
        grid=(N // _ROWBLK,),
        in_specs=[
            pl.BlockSpec((NC, _ROWBLK, GW), lambda i: (0, i, 0)),
            pl.BlockSpec((_ROWBLK, H), lambda i: (i, 0)),
            pl.BlockSpec((_ROWBLK, 1), lambda i: (i, 0)),
            pl.BlockSpec((1, 1), lambda i: (0, 0)),
            pl.BlockSpec((H, C), lambda i: (0, 0)),
            pl.BlockSpec((1, C), lambda i: (0, 0)),
        ],
        out_specs=pl.BlockSpec((_ROWBLK, C), lambda i: (i, 0)),
        out_shape=jax.ShapeDtypeStruct((N, C), jnp.float32),
    )(parts, xn, nrm, beta, w2t, b2)


def _sc_prop(xn, nrm, ei3, bvec):
    mesh = plsc.VectorSubcoreMesh(
        core_axis_name="c", subcore_axis_name="s",
        num_cores=NC, num_subcores=NS)

    @functools.partial(
        pl.kernel,
        out_type=jax.ShapeDtypeStruct((NC, N, GW), jnp.float32),
        mesh=mesh,
        scratch_types=[
            pltpu.VMEM((NB, EB), jnp.int32),
            pltpu.VMEM((NB, EB), jnp.int32),
            pltpu.VMEM((N,), jnp.float32),
            pltpu.VMEM((EB, H), jnp.float32),
            pltpu.VMEM((EB, H), jnp.float32),
            pltpu.VMEM((EB, H), jnp.float32),
            pltpu.VMEM((EB, H), jnp.float32),
            pltpu.VMEM((EB, GW), jnp.float32),
            pltpu.VMEM((EB, GW), jnp.float32),
            pltpu.VMEM((16,), jnp.float32),
            pltpu.VMEM((ZR, GW), jnp.float32),
            pltpu.VMEM_SHARED((N, GW), jnp.float32),
            pltpu.VMEM_SHARED((N, H), jnp.float32),
            pltpu.SemaphoreType.DMA,
            pltpu.SemaphoreType.DMA,
            pltpu.SemaphoreType.DMA,
            pltpu.SemaphoreType.DMA,
            pltpu.SemaphoreType.DMA,
            pltpu.SemaphoreType.DMA,
        ],
        compiler_params=pltpu.CompilerParams(
            needs_layout_passes=False, use_tc_tiling_on_sc=False),
    )
    def k(xn_hbm, nrm_hbm, ei_hbm, bvec_hbm, out_hbm,
          src_v, dst_v, nrm_v, gs0, gs1, gd0, gd1, ct0, ct1, b_v, z_v, acc_sh,
          xn_sh, sgs0, sgs1, sgd0, sgd1, sct0, sct1):
        cid = lax.axis_index("c")
        sid = lax.axis_index("s")
        wid = cid * NS + sid

        zero16 = jnp.zeros((16,), jnp.float32)

        def zbody(i, carry):
            z_v[i, 0:16] = zero16
            z_v[i, 16:32] = zero16
            return carry

        lax.fori_loop(0, ZR, zbody, 0)
        pltpu.sync_copy(z_v, acc_sh.at[pl.ds(sid * ZR, ZR)])

        @pl.when(sid == NS - 1)
        def _():
            pltpu.sync_copy(z_v.at[pl.ds(0, ZTAIL)],
                            acc_sh.at[pl.ds(NS * ZR, ZTAIL)])

        pltpu.sync_copy(bvec_hbm, b_v)
        pltpu.sync_copy(nrm_hbm, nrm_v)
        pltpu.sync_copy(ei_hbm.at[0, pl.ds(wid * NB, NB)], src_v)
        pltpu.sync_copy(ei_hbm.at[1, pl.ds(wid * NB, NB)], dst_v)
        pltpu.sync_copy(xn_hbm.at[pl.ds(sid * ZR, ZR)],
                        xn_sh.at[pl.ds(sid * ZR, ZR)])

        @pl.when(sid == NS - 1)
        def _():
            pltpu.sync_copy(xn_hbm.at[pl.ds(NS * ZR, ZTAIL)],
                            xn_sh.at[pl.ds(NS * ZR, ZTAIL)])

        plsc.subcore_barrier()

        bv16 = b_v[...]
        beta = bv16[0]
        ab = bv16[1]
        iota16 = lax.iota(jnp.int32, 16)
        diagcols = [(iota16 + kk) & 15 for kk in range(H)]
        ecols = iota16 + H
        slots = ((gs0, gd0, sgs0, sgd0), (gs1, gd1, sgs1, sgd1))
        cts = ((ct0, sct0), (ct1, sct1))

        for j in range(EB):
            ct0[j, H:GW] = zero16
            ct1[j, H:GW] = zero16

        def issue(bi, slot):
            gs, gd, sgs, sgd = slots[slot]
            pltpu.async_copy(xn_sh.at[src_v.at[bi]], gs, sgs)
            pltpu.async_copy(xn_sh.at[dst_v.at[bi]], gd, sgd)

        def wait_slot(slot):
            gs, gd, sgs, sgd = slots[slot]
            pltpu.make_async_copy(xn_hbm.at[pl.ds(0, EB)], gs, sgs).wait()
            pltpu.make_async_copy(xn_hbm.at[pl.ds(0, EB)], gd, sgd).wait()

        def scatter_start(bi, slot):
            ct, sct = cts[slot]
            pltpu.async_copy(ct, acc_sh.at[dst_v.at[bi]], sct, add=True)

        def scatter_wait(slot):
            ct, sct = cts[slot]
            pltpu.make_async_copy(ct, acc_sh.at[pl.ds(0, EB)], sct).wait()

        def compute_block(bi, slot):
            gs, gd, _, _ = slots[slot]
            ct_v, _ = cts[slot]
            for grp in range(EB // 16):
                rows = iota16 + (grp * 16)
                adiag = []
                terms = []
                for kk in range(H):
                    a = plsc.load_gather(gs, [rows, diagcols[kk]])
                    b = plsc.load_gather(gd, [rows, diagcols[kk]])
                    adiag.append(a)
                    terms.append(a * b)
                while len(terms) > 1:
                    terms = [terms[i] + terms[i + 1]
                             for i in range(0, len(terms), 2)]
                score = terms[0]
                s16 = src_v[bi, pl.ds(grp * 16, 16)]
                nsrc = plsc.load_gather(nrm_v, [s16])
                e16 = jnp.exp(beta * score - ab)
                sc16 = e16 * nsrc
                for kk in range(H):
                    plsc.store_scatter(ct_v, [rows, diagcols[kk]],
                                       adiag[kk] * sc16)
                plsc.store_scatter(ct_v, [rows, ecols], e16)

        issue(0, 0)

        def pair(g, carry):
            b0 = 2 * g
            issue(b0 + 1, 1)
            wait_slot(0)

            @pl.when(g > 0)
            def _():
                scatter_wait(0)

            compute_block(b0, 0)
            scatter_start(b0, 0)
            issue(b0 + 2, 0)
            wait_slot(1)

            @pl.when(g > 0)
            def _():
                scatter_wait(1)

            compute_block(b0 + 1, 1)
            scatter_start(b0 + 1, 1)
            return carry

        lax.fori_loop(0, (NB - 1) // 2, pair, 0)
        wait_slot(0)
        scatter_wait(0)
        compute_block(NB - 1, 0)
        scatter_start(NB - 1, 0)
        scatter_wait(0)
        scatter_wait(1)

        plsc.subcore_barrier()
        pltpu.sync_copy(acc_sh.at[pl.ds(sid * ZR, ZR)],
                        out_hbm.at[cid, pl.ds(sid * ZR, ZR)])

        @pl.when(sid == NS - 1)
        def _():
            pltpu.sync_copy(acc_sh.at[pl.ds(NS * ZR, ZTAIL)],
                            out_hbm.at[cid, pl.ds(NS * ZR, ZTAIL)])

    return k(xn, nrm, ei3, bvec)


def kernel(x, edge_index, W1, b1, W2, b2, beta2):
    x = x.astype(jnp.float32)
    ei3 = edge_index.astype(jnp.int32).reshape(2, NW * NB, EB)
    w1t = W1.astype(jnp.float32).T
    w2t = W2.astype(jnp.float32).T
    b1r = b1.astype(jnp.float32).reshape(1, H)
    b2r = b2.astype(jnp.float32).reshape(1, C)
    beta2f = beta2.astype(jnp.float32)

    xn0, nrm0 = _tc_in(x, w1t, b1r)
    bvec1 = jnp.asarray([1.0, 1.0] + [0.0] * 14, dtype=jnp.float32)
    parts1 = _sc_prop(xn0, nrm0.reshape(N), ei3, bvec1)
    xn1, nrm1 = _tc_mid(parts1, xn0, nrm0)
    bvec2 = jnp.concatenate(
        [beta2f, jnp.abs(beta2f), jnp.zeros((14,), jnp.float32)])
    parts2 = _sc_prop(xn1, nrm1.reshape(N), ei3, bvec2)
    return _tc_out(parts2, xn1, nrm1,
                   beta2f.reshape(1, 1), w2t, b2r)


# speedup vs baseline: 1.3875x; 1.0655x over previous
import functools

import jax
import jax.numpy as jnp
from jax import lax
from jax.experimental import pallas as pl
from jax.experimental.pallas import tpu as pltpu
from jax.experimental.pallas import tpu_sc as plsc

N = 10000
E = 320000
D = 128
H = 16
C = 40

NC = 2
NS = 16
NW = NC * NS
EPW = E // NW
EB = 80
NB = EPW // EB
GW = 24
ZR = 624
ZTAIL = N - NS * ZR

_ROWBLK = 1000


def _tc_in_body(x_ref, w1t_ref, b1_ref, xn_ref, nrm_ref):
    h = jnp.dot(x_ref[...], w1t_ref[...], preferred_element_type=jnp.float32)
    h = jnp.maximum(h + b1_ref[...], 0.0)
    n2 = jnp.sum(h * h, axis=1, keepdims=True)
    norm = jnp.sqrt(n2)
    rnorm = 1.0 / jnp.maximum(norm, 1e-12)
    xn_ref[...] = h * rnorm
    nrm_ref[...] = norm


def _tc_out_body(p1_ref, p2_ref, xn0_ref, nrm0_ref, beta_ref, w2t_ref,
                 b2_ref, out_ref):
    beta = beta_ref[0, 0]
    xn0 = xn0_ref[...]
    c0 = jnp.sum(xn0 * xn0, axis=1, keepdims=True)
    es1 = jnp.exp(c0 - 1.0)
    pa1 = p1_ref[0]
    pb1 = p1_ref[1]
    num1 = pa1[:, 0:H] + pb1[:, 0:H] + xn0 * nrm0_ref[...] * es1
    den1 = (jnp.sum(pa1[:, H:GW], axis=1, keepdims=True)
            + jnp.sum(pb1[:, H:GW], axis=1, keepdims=True) + es1)
    h1 = num1 / den1
    n2 = jnp.sum(h1 * h1, axis=1, keepdims=True)
    rnorm1 = 1.0 / jnp.maximum(jnp.sqrt(n2), 1e-12)
    c1 = n2 * rnorm1 * rnorm1
    es2 = jnp.exp(beta * c1 - jnp.abs(beta))
    pa2 = p2_ref[0]
    pb2 = p2_ref[1]
    num2 = pa2[:, 0:H] + pb2[:, 0:H] + h1 * es2
    den2 = (jnp.sum(pa2[:, H:GW], axis=1, keepdims=True)
            + jnp.sum(pb2[:, H:GW], axis=1, keepdims=True) + es2)
    h2 = num2 / den2
    logits = jnp.dot(h2, w2t_ref[...], preferred_element_type=jnp.float32)
    logits = logits + b2_ref[...]
    m = jnp.max(logits, axis=1, keepdims=True)
    lse = jnp.log(jnp.sum(jnp.exp(logits - m), axis=1, keepdims=True)) + m
    out_ref[...] = logits - lse


def _tc_in(x, w1t, b1):
    return pl.pallas_call(
        _tc_in_body,
        grid=(N // _ROWBLK,),
        in_specs=[
            pl.BlockSpec((_ROWBLK, D), lambda i: (i, 0)),
            pl.BlockSpec((D, H), lambda i: (0, 0)),
            pl.BlockSpec((1, H), lambda i: (0, 0)),
        ],
        out_specs=[
            pl.BlockSpec((_ROWBLK, H), lambda i: (i, 0)),
            pl.BlockSpec((_ROWBLK, 1), lambda i: (i, 0)),
        ],
        out_shape=[
            jax.ShapeDtypeStruct((N, H), jnp.float32),
            jax.ShapeDtypeStruct((N, 1), jnp.float32),
        ],
    )(x, w1t, b1)


def _tc_out(parts1, parts2, xn0, nrm0, beta, w2t, b2):
    return pl.pallas_call(
        _tc_out_body,
        grid=(N // _ROWBLK,),
        in_specs=[
            pl.BlockSpec((NC, _ROWBLK, GW), lambda i: (0, i, 0)),
            pl.BlockSpec((NC, _ROWBLK, GW), lambda i: (0, i, 0)),
            pl.BlockSpec((_ROWBLK, H), lambda i: (i, 0)),
            pl.BlockSpec((_ROWBLK, 1), lambda i: (i, 0)),
            pl.BlockSpec((1, 1), lambda i: (0, 0)),
            pl.BlockSpec((H, C), lambda i: (0, 0)),
            pl.BlockSpec((1, C), lambda i: (0, 0)),
        ],
        out_specs=pl.BlockSpec((_ROWBLK, C), lambda i: (i, 0)),
        out_shape=jax.ShapeDtypeStruct((N, C), jnp.float32),
    )(parts1, parts2, xn0, nrm0, beta, w2t, b2)


def _tree(terms):
    terms = list(terms)
    while len(terms) > 1:
        terms = [terms[i] + terms[i + 1] for i in range(0, len(terms), 2)]
    return terms[0]


def _rsqrt16(x):
    xi = plsc.bitcast(x, jnp.int32)
    yi = jnp.int32(0x5F3759DF) - lax.shift_right_arithmetic(xi, 1)
    y = plsc.bitcast(yi, jnp.float32)
    for _ in range(3):
        y = y * (1.5 - 0.5 * x * y * y)
    return y


ZB = ZR + ZTAIL


def _sc_prop(xn, nrm, ei3, bvec, parts=None):
    combine = parts is not None
    mesh = plsc.VectorSubcoreMesh(
        core_axis_name="c", subcore_axis_name="s",
        num_cores=NC, num_subcores=NS)

    scratch = [
        pltpu.VMEM((NB, EB), jnp.int32),
        pltpu.VMEM((NB, EB), jnp.int32),
        pltpu.VMEM((N,), jnp.float32),
        pltpu.VMEM((EB, H), jnp.float32),
        pltpu.VMEM((EB, H), jnp.float32),
        pltpu.VMEM((EB, H), jnp.float32),
        pltpu.VMEM((EB, H), jnp.float32),
        pltpu.VMEM((EB, GW), jnp.float32),
        pltpu.VMEM((EB, GW), jnp.float32),
        pltpu.VMEM((16,), jnp.float32),
        pltpu.VMEM((ZB, GW), jnp.float32),
        pltpu.VMEM_SHARED((N, GW), jnp.float32),
        pltpu.VMEM_SHARED((N, H), jnp.float32),
    ]
    if combine:
        scratch += [
            pltpu.VMEM((ZB, GW), jnp.float32),
            pltpu.VMEM((ZB, H), jnp.float32),
            pltpu.VMEM((ZB, H), jnp.float32),
            pltpu.VMEM((ZB,), jnp.float32),
            pltpu.VMEM((ZB,), jnp.float32),
            pltpu.VMEM_SHARED((N,), jnp.float32),
        ]
    scratch += [pltpu.SemaphoreType.DMA] * 6

    @functools.partial(
        pl.kernel,
        out_type=jax.ShapeDtypeStruct((NC, N, GW), jnp.float32),
        mesh=mesh,
        scratch_types=scratch,
        compiler_params=pltpu.CompilerParams(
            needs_layout_passes=False, use_tc_tiling_on_sc=False),
    )
    def k(*refs):
        if combine:
            (xn_hbm, nrm_hbm, ei_hbm, bvec_hbm, parts_hbm, out_hbm,
             src_v, dst_v, nrm_v, gs0, gs1, gd0, gd1, ct0, ct1, b_v, z_v,
             acc_sh, xn_sh, pb_v, x0_v, xn1_v, n0_v, n1_v, nrm_sh,
             sgs0, sgs1, sgd0, sgd1, sct0, sct1) = refs
        else:
            (xn_hbm, nrm_hbm, ei_hbm, bvec_hbm, out_hbm,
             src_v, dst_v, nrm_v, gs0, gs1, gd0, gd1, ct0, ct1, b_v, z_v,
             acc_sh, xn_sh,
             sgs0, sgs1, sgd0, sgd1, sct0, sct1) = refs
        cid = lax.axis_index("c")
        sid = lax.axis_index("s")
        wid = cid * NS + sid

        zero16 = jnp.zeros((16,), jnp.float32)

        def zbody(i, carry):
            z_v[i, 0:16] = zero16
            z_v[i, 8:24] = zero16
            return carry

        lax.fori_loop(0, ZR, zbody, 0)
        pltpu.sync_copy(z_v.at[pl.ds(0, ZR)], acc_sh.at[pl.ds(sid * ZR, ZR)])

        @pl.when(sid == NS - 1)
        def _():
            pltpu.sync_copy(z_v.at[pl.ds(0, ZTAIL)],
                            acc_sh.at[pl.ds(NS * ZR, ZTAIL)])

        pltpu.sync_copy(bvec_hbm, b_v)
        pltpu.sync_copy(ei_hbm.at[0, pl.ds(wid * NB, NB)], src_v)
        pltpu.sync_copy(ei_hbm.at[1, pl.ds(wid * NB, NB)], dst_v)

        bv16 = b_v[...]
        beta = bv16[0]
        ab = bv16[1]
        iota16 = lax.iota(jnp.int32, 16)
        diagcols = [(iota16 + kk) & 15 for kk in range(H)]
        ecols = H + (iota16 & 7)
        slots = ((gs0, gd0, sgs0, sgd0), (gs1, gd1, sgs1, sgd1))
        cts = ((ct0, sct0), (ct1, sct1))

        if combine:
            pltpu.sync_copy(xn_hbm.at[pl.ds(sid * ZR, ZB)], x0_v)
            pltpu.sync_copy(nrm_hbm.at[pl.ds(sid * ZR, ZB)], n0_v)
            pltpu.sync_copy(parts_hbm.at[0, pl.ds(sid * ZR, ZB)], z_v)
            pltpu.sync_copy(parts_hbm.at[1, pl.ds(sid * ZR, ZB)], pb_v)
            dch = [H + ((iota16 + kk) & 7) for kk in range(GW - H)]

            def cg(g, carry):
                rows = iota16 + g * 16
                x0d = [plsc.load_gather(x0_v, [rows, diagcols[kk]])
                       for kk in range(H)]
                c0 = _tree([v * v for v in x0d])
                es = jnp.exp(c0 - 1.0)
                w16 = plsc.load_gather(n0_v, [rows]) * es
                num_d = []
                for kk in range(H):
                    palo = plsc.load_gather(z_v, [rows, diagcols[kk]])
                    pblo = plsc.load_gather(pb_v, [rows, diagcols[kk]])
                    num_d.append(palo + pblo + x0d[kk] * w16)
                den_t = [plsc.load_gather(z_v, [rows, dch[kk]])
                         + plsc.load_gather(pb_v, [rows, dch[kk]])
                         for kk in range(GW - H)]
                rden = 1.0 / (_tree(den_t) + es)
                h1d = [v * rden for v in num_d]
                n2 = _tree([v * v for v in h1d])
                rn = _rsqrt16(jnp.maximum(n2, 1e-24))
                for kk in range(H):
                    plsc.store_scatter(xn1_v, [rows, diagcols[kk]],
                                       h1d[kk] * rn)
                n1_v[pl.ds(g * 16, 16)] = n2 * rn
                return carry

            lax.fori_loop(0, ZB // 16, cg, 0)
            pltpu.sync_copy(xn1_v.at[pl.ds(0, ZR)],
                            xn_sh.at[pl.ds(sid * ZR, ZR)])
            pltpu.sync_copy(n1_v.at[pl.ds(0, ZR)],
                            nrm_sh.at[pl.ds(sid * ZR, ZR)])

            @pl.when(sid == NS - 1)
            def _():
                pltpu.sync_copy(xn1_v.at[pl.ds(ZR, ZTAIL)],
                                xn_sh.at[pl.ds(NS * ZR, ZTAIL)])
                pltpu.sync_copy(n1_v.at[pl.ds(ZR, ZTAIL)],
                                nrm_sh.at[pl.ds(NS * ZR, ZTAIL)])

            plsc.subcore_barrier()
            pltpu.sync_copy(nrm_sh, nrm_v)
        else:
            pltpu.sync_copy(nrm_hbm, nrm_v)
            pltpu.sync_copy(xn_hbm.at[pl.ds(sid * ZR, ZR)],
                            xn_sh.at[pl.ds(sid * ZR, ZR)])

            @pl.when(sid == NS - 1)
            def _():
                pltpu.sync_copy(xn_hbm.at[pl.ds(NS * ZR, ZTAIL)],
                                xn_sh.at[pl.ds(NS * ZR, ZTAIL)])

            plsc.subcore_barrier()

        for j in range(EB):
            ct0[j, 8:24] = zero16
            ct1[j, 8:24] = zero16

        def issue(bi, slot):
            gs, gd, sgs, sgd = slots[slot]
            pltpu.async_copy(xn_sh.at[src_v.at[bi]], gs, sgs)
            pltpu.async_copy(xn_sh.at[dst_v.at[bi]], gd, sgd)

        def wait_slot(slot):
            gs, gd, sgs, sgd = slots[slot]
            pltpu.make_async_copy(xn_hbm.at[pl.ds(0, EB)], gs, sgs).wait()
            pltpu.make_async_copy(xn_hbm.at[pl.ds(0, EB)], gd, sgd).wait()

        def scatter_start(bi, slot):
            ct, sct = cts[slot]
            pltpu.async_copy(ct, acc_sh.at[dst_v.at[bi]], sct, add=True)

        def scatter_wait(slot):
            ct, sct = cts[slot]
            pltpu.make_async_copy(ct, acc_sh.at[pl.ds(0, EB)], sct).wait()

        def compute_block(bi, slot):
            gs, gd, _, _ = slots[slot]
            ct_v, _ = cts[slot]
            for grp in range(EB // 16):
                rows = iota16 + (grp * 16)
                adiag = []
                terms = []
                for kk in range(H):
                    a = plsc.load_gather(gs, [rows, diagcols[kk]])
                    b = plsc.load_gather(gd, [rows, diagcols[kk]])
                    adiag.append(a)
                    terms.append(a * b)
                while len(terms) > 1:
                    terms = [terms[i] + terms[i + 1]
                             for i in range(0, len(terms), 2)]
                score = terms[0]
                s16 = src_v[bi, pl.ds(grp * 16, 16)]
                nsrc = plsc.load_gather(nrm_v, [s16])
                e16 = jnp.exp(beta * score - ab)
                sc16 = e16 * nsrc
                for kk in range(H):
                    plsc.store_scatter(ct_v, [rows, diagcols[kk]],
                                       adiag[kk] * sc16)
                plsc.store_scatter(ct_v, [rows, ecols], e16)

        issue(0, 0)

        def pair(g, carry):
            b0 = 2 * g
            issue(b0 + 1, 1)
            wait_slot(0)

            @pl.when(g > 0)
            def _():
                scatter_wait(0)

            compute_block(b0, 0)
            scatter_start(b0, 0)
            issue(b0 + 2, 0)
            wait_slot(1)

            @pl.when(g > 0)
            def _():
                scatter_wait(1)

            compute_block(b0 + 1, 1)
            scatter_start(b0 + 1, 1)
            return carry

        lax.fori_loop(0, (NB - 1) // 2, pair, 0)
        wait_slot(0)
        scatter_wait(0)
        compute_block(NB - 1, 0)
        scatter_start(NB - 1, 0)
        scatter_wait(0)
        scatter_wait(1)

        plsc.subcore_barrier()
        pltpu.sync_copy(acc_sh.at[pl.ds(sid * ZR, ZR)],
                        out_hbm.at[cid, pl.ds(sid * ZR, ZR)])

        @pl.when(sid == NS - 1)
        def _():
            pltpu.sync_copy(acc_sh.at[pl.ds(NS * ZR, ZTAIL)],
                            out_hbm.at[cid, pl.ds(NS * ZR, ZTAIL)])

    if combine:
        return k(xn, nrm, ei3, bvec, parts)
    return k(xn, nrm, ei3, bvec)


def kernel(x, edge_index, W1, b1, W2, b2, beta2):
    x = x.astype(jnp.float32)
    ei3 = edge_index.astype(jnp.int32).reshape(2, NW * NB, EB)
    w1t = W1.astype(jnp.float32).T
    w2t = W2.astype(jnp.float32).T
    b1r = b1.astype(jnp.float32).reshape(1, H)
    b2r = b2.astype(jnp.float32).reshape(1, C)
    beta2f = beta2.astype(jnp.float32)

    xn0, nrm0 = _tc_in(x, w1t, b1r)
    nrm0f = nrm0.reshape(N)
    bvec1 = jnp.asarray([1.0, 1.0] + [0.0] * 14, dtype=jnp.float32)
    parts1 = _sc_prop(xn0, nrm0f, ei3, bvec1)
    bvec2 = jnp.concatenate(
        [beta2f, jnp.abs(beta2f), jnp.zeros((14,), jnp.float32)])
    parts2 = _sc_prop(xn0, nrm0f, ei3, bvec2, parts=parts1)
    return _tc_out(parts1, parts2, xn0, nrm0,
                   beta2f.reshape(1, 1), w2t, b2r)
